# baseline (speedup 1.0000x reference)
import functools

import jax
import jax.numpy as jnp
from jax import lax
from jax.experimental import pallas as pl
from jax.experimental.pallas import tpu as pltpu
from jax.experimental.pallas import tpu_sc as plsc

D = 64
L = 16
NC, NS = 2, 16
NW = NC * NS
B = 4096
SEQ = 200
SEQ_PAD = 208
ROWS_PER_W = B // NW
CHUNKS = SEQ_PAD // L
G0, G1 = 128, SEQ_PAD - 128

_mesh = plsc.VectorSubcoreMesh(
    core_axis_name="c", subcore_axis_name="s", num_cores=NC, num_subcores=NS
)


@functools.partial(
    pl.kernel,
    out_type=(
        jax.ShapeDtypeStruct((B, D), jnp.float32),
        jax.ShapeDtypeStruct((B, L), jnp.float32),
    ),
    mesh=_mesh,
    compiler_params=pltpu.CompilerParams(use_tc_tiling_on_sc=False),
    scratch_types=[
        pltpu.VMEM((ROWS_PER_W, SEQ_PAD), jnp.int32),
        pltpu.VMEM((2, SEQ_PAD, D), jnp.float32),
        pltpu.VMEM((ROWS_PER_W, D), jnp.float32),
        pltpu.VMEM((ROWS_PER_W, L), jnp.float32),
        pltpu.SemaphoreType.DMA,
        pltpu.SemaphoreType.DMA,
    ],
)
def _sc_pool(ids_hbm, table_hbm, sums_hbm, counts_hbm,
             ids_v, rows_v, sums_v, counts_v, sem0, sem1):
    sems = (sem0, sem1)
    wid = lax.axis_index("s") * NC + lax.axis_index("c")
    base = wid * ROWS_PER_W

    pltpu.sync_copy(ids_hbm.at[pl.ds(base, ROWS_PER_W)], ids_v)

    def start_gather(row, buf):
        pltpu.async_copy(
            table_hbm.at[ids_v.at[row, pl.ds(0, G0)]],
            rows_v.at[buf, pl.ds(0, G0)], sems[buf])
        pltpu.async_copy(
            table_hbm.at[ids_v.at[row, pl.ds(G0, G1)]],
            rows_v.at[buf, pl.ds(G0, G1)], sems[buf])

    def wait_gather(buf):
        pltpu.make_async_copy(
            table_hbm.at[pl.ds(0, G0)], rows_v.at[buf, pl.ds(0, G0)],
            sems[buf]).wait()
        pltpu.make_async_copy(
            table_hbm.at[pl.ds(0, G1)], rows_v.at[buf, pl.ds(G0, G1)],
            sems[buf]).wait()

    start_gather(0, 0)

    def do_row(row, buf):
        @pl.when(row + 1 < ROWS_PER_W)
        def _():
            start_gather(row + 1, 1 - buf)

        wait_gather(buf)

        z = jnp.zeros((L,), jnp.float32)

        def acc_chunk(c, carry):
            accs = list(carry[:8])
            cnt = carry[8]
            ids16 = ids_v[row, pl.ds(c * L, L)]
            cnt = cnt + jnp.where(ids16 != 0, 1.0, 0.0)
            for i in range(L):
                j = c * L + i
                par = i & 1
                for k in range(4):
                    accs[4 * par + k] = (
                        accs[4 * par + k] + rows_v[buf, j, pl.ds(k * L, L)])
            return tuple(accs) + (cnt,)

        res = lax.fori_loop(0, CHUNKS, acc_chunk, (z,) * 9)
        for k in range(4):
            sums_v[row, pl.ds(k * L, L)] = res[k] + res[4 + k]
        counts_v[row, pl.ds(0, L)] = res[8]

    @pl.loop(0, ROWS_PER_W, step=2)
    def _(r):
        do_row(r, 0)
        do_row(r + 1, 1)

    pltpu.sync_copy(sums_v, sums_hbm.at[pl.ds(base, ROWS_PER_W)])
    pltpu.sync_copy(counts_v, counts_hbm.at[pl.ds(base, ROWS_PER_W)])


def _tc_head(sums_ref, counts_ref, w_ref, b_ref, out_ref):
    cnt = jnp.sum(counts_ref[...], axis=1, keepdims=True)
    se = sums_ref[...] / (cnt + 1e-8)
    out_ref[...] = lax.dot_general(
        se, w_ref[...], (((1,), (1,)), ((), ())),
        preferred_element_type=jnp.float32) + b_ref[...]


def kernel(input_ids, table, W, b):
    ids = input_ids.astype(jnp.int32)
    ids_pad = jnp.pad(ids, ((0, 0), (0, SEQ_PAD - SEQ)))
    sums, counts = _sc_pool(ids_pad, table)
    logits = pl.pallas_call(
        _tc_head,
        out_shape=jax.ShapeDtypeStruct((B, W.shape[0]), jnp.float32),
    )(sums, counts, W, b.reshape(1, -1))
    return logits


# speedup vs baseline: 1.8396x; 1.8396x over previous
import functools

import jax
import jax.numpy as jnp
from jax import lax
from jax.experimental import pallas as pl
from jax.experimental.pallas import tpu as pltpu
from jax.experimental.pallas import tpu_sc as plsc

D = 64
L = 16
NC, NS = 2, 16
NW = NC * NS
B = 4096
SEQ = 200
SEQ_PAD = 208
ROWS_PER_W = B // NW
TOK_W = ROWS_PER_W * SEQ_PAD
CHUNKS = SEQ_PAD // L
G0, G1 = 128, SEQ_PAD - 128
VOCAB = 1000000
PROJ_BLK = 16384
PROJ_GRID = -(-VOCAB // PROJ_BLK)
VOCAB_PAD = PROJ_GRID * PROJ_BLK

_mesh = plsc.VectorSubcoreMesh(
    core_axis_name="c", subcore_axis_name="s", num_cores=NC, num_subcores=NS
)


def _tc_project(tab_ref, w_ref, p_ref):
    t = tab_ref[...]
    w = w_ref[...]
    p_ref[...] = lax.dot_general(
        w, t, (((1,), (1,)), ((), ())),
        preferred_element_type=jnp.float32)


@functools.partial(
    pl.kernel,
    out_type=jax.ShapeDtypeStruct((B, 3 * L), jnp.float32),
    mesh=_mesh,
    compiler_params=pltpu.CompilerParams(use_tc_tiling_on_sc=False),
    scratch_types=[
        pltpu.VMEM((TOK_W,), jnp.int32),
        pltpu.VMEM((2, SEQ_PAD), jnp.float32),
        pltpu.VMEM((2, SEQ_PAD), jnp.float32),
        pltpu.VMEM((ROWS_PER_W, 3 * L), jnp.float32),
        pltpu.SemaphoreType.DMA,
        pltpu.SemaphoreType.DMA,
    ],
)
def _sc_pool(ids_hbm, p0_hbm, p1_hbm, parts_hbm,
             ids_v, g0_v, g1_v, parts_v, sem0, sem1):
    sems = (sem0, sem1)
    wid = lax.axis_index("s") * NC + lax.axis_index("c")
    base = wid * ROWS_PER_W

    pltpu.sync_copy(ids_hbm.at[pl.ds(wid * TOK_W, TOK_W)], ids_v)

    def start_gather(row, buf):
        off = row * SEQ_PAD
        idx_a = ids_v.at[pl.ds(off, G0)]
        idx_b = ids_v.at[pl.ds(off + G0, G1)]
        sem = sems[buf]
        pltpu.async_copy(p0_hbm.at[idx_a], g0_v.at[buf, pl.ds(0, G0)], sem)
        pltpu.async_copy(p0_hbm.at[idx_b], g0_v.at[buf, pl.ds(G0, G1)], sem)
        pltpu.async_copy(p1_hbm.at[idx_a], g1_v.at[buf, pl.ds(0, G0)], sem)
        pltpu.async_copy(p1_hbm.at[idx_b], g1_v.at[buf, pl.ds(G0, G1)], sem)

    def wait_gather(buf):
        sem = sems[buf]
        pltpu.make_async_copy(
            p0_hbm.at[pl.ds(0, G0)], g0_v.at[buf, pl.ds(0, G0)], sem).wait()
        pltpu.make_async_copy(
            p0_hbm.at[pl.ds(0, G1)], g0_v.at[buf, pl.ds(G0, G1)], sem).wait()
        pltpu.make_async_copy(
            p1_hbm.at[pl.ds(0, G0)], g1_v.at[buf, pl.ds(0, G0)], sem).wait()
        pltpu.make_async_copy(
            p1_hbm.at[pl.ds(0, G1)], g1_v.at[buf, pl.ds(G0, G1)], sem).wait()

    start_gather(0, 0)

    def do_row(row, buf):
        @pl.when(row + 1 < ROWS_PER_W)
        def _():
            start_gather(row + 1, 1 - buf)

        wait_gather(buf)

        z = jnp.zeros((L,), jnp.float32)

        def acc_chunk(c, carry):
            a0, a1, cnt = carry
            a0 = a0 + g0_v[buf, pl.ds(c * L, L)]
            a1 = a1 + g1_v[buf, pl.ds(c * L, L)]
            ids16 = ids_v[pl.ds(row * SEQ_PAD + c * L, L)]
            cnt = cnt + jnp.where(ids16 != 0, 1.0, 0.0)
            return a0, a1, cnt

        a0, a1, cnt = lax.fori_loop(0, CHUNKS, acc_chunk, (z, z, z))
        parts_v[row, pl.ds(0, L)] = a0
        parts_v[row, pl.ds(L, L)] = a1
        parts_v[row, pl.ds(2 * L, L)] = cnt

    @pl.loop(0, ROWS_PER_W, step=2)
    def _(r):
        do_row(r, 0)
        do_row(r + 1, 1)

    pltpu.sync_copy(parts_v, parts_hbm.at[pl.ds(base, ROWS_PER_W)])


def _tc_head(parts_ref, b_ref, out_ref):
    p = parts_ref[...]
    c0 = jnp.sum(p[:, 0:L], axis=1, keepdims=True)
    c1 = jnp.sum(p[:, L:2 * L], axis=1, keepdims=True)
    cnt = jnp.sum(p[:, 2 * L:3 * L], axis=1, keepdims=True)
    se = jnp.concatenate([c0, c1], axis=1) / (cnt + 1e-8)
    out_ref[...] = se + b_ref[...]


def kernel(input_ids, table, W, b):
    ids = input_ids.astype(jnp.int32)
    ids_flat = jnp.pad(ids, ((0, 0), (0, SEQ_PAD - SEQ))).reshape(-1)
    p01 = pl.pallas_call(
        _tc_project,
        grid=(PROJ_GRID,),
        in_specs=[
            pl.BlockSpec((PROJ_BLK, D), lambda i: (i, 0)),
            pl.BlockSpec((2, D), lambda i: (0, 0)),
        ],
        out_specs=pl.BlockSpec((2, PROJ_BLK), lambda i: (0, i)),
        out_shape=jax.ShapeDtypeStruct((2, VOCAB_PAD), jnp.float32),
    )(table, W)
    parts = _sc_pool(ids_flat, p01[0], p01[1])
    logits = pl.pallas_call(
        _tc_head,
        out_shape=jax.ShapeDtypeStruct((B, W.shape[0]), jnp.float32),
    )(parts, b.reshape(1, -1))
    return logits
